# Initial kernel scaffold; baseline (speedup 1.0000x reference)
import functools

import jax
import jax.numpy as jnp
from jax import lax
from jax.experimental import pallas as pl
from jax.experimental.pallas import tpu as pltpu
from jax.experimental.pallas import tpu_sc as plsc

NUM_SC = 2
NUM_SUBCORES = 16
LANES = 16


def _build_dense(vals, cols, n, m):
    kp = vals.shape[1]
    nw = NUM_SC * NUM_SUBCORES
    rpt = n // nw
    nchunk = kp // LANES

    @functools.partial(
        pl.kernel,
        out_type=jax.ShapeDtypeStruct((n, m), jnp.float32),
        mesh=plsc.VectorSubcoreMesh(core_axis_name="c", subcore_axis_name="s"),
        scratch_types=[
            pltpu.VMEM((rpt, kp), jnp.float32),
            pltpu.VMEM((rpt, kp), jnp.int32),
            pltpu.VMEM((m,), jnp.float32),
        ],
    )
    def scatter_kernel(vals_hbm, cols_hbm, wd_hbm, vals_v, cols_v, rowbuf):
        wid = lax.axis_index("s") * NUM_SC + lax.axis_index("c")
        base = wid * rpt
        pltpu.sync_copy(vals_hbm.at[pl.ds(base, rpt)], vals_v)
        pltpu.sync_copy(cols_hbm.at[pl.ds(base, rpt)], cols_v)

        zero16 = jnp.zeros((LANES,), jnp.float32)

        def zinit(i, carry):
            rowbuf[pl.ds(i * LANES, LANES)] = zero16
            return carry

        lax.fori_loop(0, m // LANES, zinit, 0)

        def row_body(r, carry):
            @pl.when(r > 0)
            def _():
                for c in range(nchunk):
                    idx = cols_v[r - 1, pl.ds(c * LANES, LANES)]
                    plsc.store_scatter(rowbuf, [idx], zero16)

            for c in range(nchunk):
                idx = cols_v[r, pl.ds(c * LANES, LANES)]
                v = vals_v[r, pl.ds(c * LANES, LANES)]
                plsc.addupdate_scatter(rowbuf, [idx], v)

            pltpu.sync_copy(rowbuf, wd_hbm.at[base + r])
            return carry

        lax.fori_loop(0, rpt, row_body, 0)

    return scatter_kernel(vals, cols)


def _matmul_body(x_ref, wd_ref, bias_ref, out_ref):
    acc = lax.dot_general(
        x_ref[...], wd_ref[...],
        dimension_numbers=(((1,), (1,)), ((), ())),
        preferred_element_type=jnp.float32,
    )
    nb = out_ref.shape[1]
    j = pl.program_id(0)
    out_ref[...] = acc + bias_ref[pl.ds(j * nb, nb)][None, :]


def kernel(input, W_val, W_cols, bias):
    b, m = input.shape
    n, k = W_val.shape
    kp = ((k + LANES - 1) // LANES) * LANES
    vals = jnp.pad(W_val, ((0, 0), (0, kp - k)))
    cols = jnp.pad(W_cols.astype(jnp.int32), ((0, 0), (0, kp - k)))

    wd = _build_dense(vals, cols, n, m)

    nb = 512
    out = pl.pallas_call(
        _matmul_body,
        grid=(n // nb,),
        in_specs=[
            pl.BlockSpec((b, m), lambda i: (0, 0)),
            pl.BlockSpec((nb, m), lambda i: (i, 0)),
            pl.BlockSpec((n,), lambda i: (0,)),
        ],
        out_specs=pl.BlockSpec((b, nb), lambda i: (0, i)),
        out_shape=jax.ShapeDtypeStruct((b, n), jnp.float32),
        compiler_params=pltpu.CompilerParams(
            dimension_semantics=("arbitrary",)),
    )(input, wd, bias)
    return out


# speedup vs baseline: 10.7956x; 10.7956x over previous
import functools

import jax
import jax.numpy as jnp
from jax import lax
from jax.experimental import pallas as pl
from jax.experimental.pallas import tpu as pltpu
from jax.experimental.pallas import tpu_sc as plsc

NUM_SC = 2
NUM_SUBCORES = 16
LANES = 16


def _build_dense(vals, cols, n, m):
    kp = vals.shape[1]
    nw = NUM_SC * NUM_SUBCORES
    rpt = n // nw
    nchunk = kp // LANES

    @functools.partial(
        pl.kernel,
        out_type=jax.ShapeDtypeStruct((n, m), jnp.float32),
        mesh=plsc.VectorSubcoreMesh(core_axis_name="c", subcore_axis_name="s"),
        compiler_params=pltpu.CompilerParams(needs_layout_passes=False),
        scratch_types=[
            pltpu.VMEM((rpt, kp), jnp.float32),
            pltpu.VMEM((rpt, kp), jnp.int32),
            pltpu.VMEM((m,), jnp.float32),
        ],
    )
    def scatter_kernel(vals_hbm, cols_hbm, wd_hbm, vals_v, cols_v, rowbuf):
        wid = lax.axis_index("s") * NUM_SC + lax.axis_index("c")
        base = wid * rpt
        pltpu.sync_copy(vals_hbm.at[pl.ds(base, rpt)], vals_v)
        pltpu.sync_copy(cols_hbm.at[pl.ds(base, rpt)], cols_v)

        zero16 = jnp.zeros((LANES,), jnp.float32)

        def zinit(i, carry):
            rowbuf[pl.ds(i * LANES, LANES)] = zero16
            return carry

        lax.fori_loop(0, m // LANES, zinit, 0)

        def row_body(r, carry):
            @pl.when(r > 0)
            def _():
                for c in range(nchunk):
                    idx = cols_v[r - 1, pl.ds(c * LANES, LANES)]
                    plsc.store_scatter(rowbuf, [idx], zero16)

            for c in range(nchunk):
                idx = cols_v[r, pl.ds(c * LANES, LANES)]
                v = vals_v[r, pl.ds(c * LANES, LANES)]
                plsc.addupdate_scatter(rowbuf, [idx], v)

            pltpu.sync_copy(rowbuf, wd_hbm.at[base + r])
            return carry

        lax.fori_loop(0, rpt, row_body, 0)

    return scatter_kernel(vals, cols)


def _matmul_body(x_ref, wd_ref, bias_ref, out_ref):
    acc = lax.dot_general(
        x_ref[...], wd_ref[...],
        dimension_numbers=(((1,), (1,)), ((), ())),
        preferred_element_type=jnp.float32,
    )
    nb = out_ref.shape[1]
    j = pl.program_id(0)
    out_ref[...] = acc + bias_ref[pl.ds(j * nb, nb)][None, :]


def kernel(input, W_val, W_cols, bias):
    b, m = input.shape
    n, k = W_val.shape
    kp = ((k + LANES - 1) // LANES) * LANES
    vals = jnp.pad(W_val, ((0, 0), (0, kp - k)))
    cols = jnp.pad(W_cols.astype(jnp.int32), ((0, 0), (0, kp - k)))

    wd = _build_dense(vals, cols, n, m)

    nb = 512
    out = pl.pallas_call(
        _matmul_body,
        grid=(n // nb,),
        in_specs=[
            pl.BlockSpec((b, m), lambda i: (0, 0)),
            pl.BlockSpec((nb, m), lambda i: (i, 0)),
            pl.BlockSpec((n,), lambda i: (0,)),
        ],
        out_specs=pl.BlockSpec((b, nb), lambda i: (0, i)),
        out_shape=jax.ShapeDtypeStruct((b, n), jnp.float32),
        compiler_params=pltpu.CompilerParams(
            dimension_semantics=("arbitrary",)),
    )(input, wd, bias)
    return out


# speedup vs baseline: 12.4189x; 1.1504x over previous
import functools

import jax
import jax.numpy as jnp
from jax import lax
from jax.experimental import pallas as pl
from jax.experimental.pallas import tpu as pltpu
from jax.experimental.pallas import tpu_sc as plsc

NUM_SC = 2
NUM_SUBCORES = 16
LANES = 16


def _build_dense(vals, cols, n, m):
    kp = vals.shape[1]
    nw = NUM_SC * NUM_SUBCORES
    rpt = n // nw
    nchunk = kp // LANES

    grp = 4
    ngroups = rpt // grp

    @functools.partial(
        pl.kernel,
        out_type=jax.ShapeDtypeStruct((n, m), jnp.float32),
        mesh=plsc.VectorSubcoreMesh(core_axis_name="c", subcore_axis_name="s"),
        compiler_params=pltpu.CompilerParams(needs_layout_passes=False),
        scratch_types=[
            pltpu.VMEM((rpt, kp), jnp.float32),
            pltpu.VMEM((rpt, kp), jnp.int32),
            pltpu.VMEM((grp, m), jnp.float32),
            pltpu.VMEM((grp, m), jnp.float32),
            pltpu.SemaphoreType.DMA,
            pltpu.SemaphoreType.DMA,
        ],
    )
    def scatter_kernel(vals_hbm, cols_hbm, wd_hbm, vals_v, cols_v,
                       buf0, buf1, sem0, sem1):
        wid = lax.axis_index("s") * NUM_SC + lax.axis_index("c")
        base = wid * rpt
        pltpu.sync_copy(vals_hbm.at[pl.ds(base, rpt)], vals_v)
        pltpu.sync_copy(cols_hbm.at[pl.ds(base, rpt)], cols_v)

        zero16 = jnp.zeros((LANES,), jnp.float32)
        bufs = (buf0, buf1)
        sems = (sem0, sem1)

        def zinit(i, carry):
            for gg in range(grp):
                buf0[gg, pl.ds(i * LANES, LANES)] = zero16
                buf1[gg, pl.ds(i * LANES, LANES)] = zero16
            return carry

        lax.fori_loop(0, m // LANES, zinit, 0)

        def pair_body(t, carry):
            for bsel in range(2):
                g = t * 2 + bsel
                buf = bufs[bsel]
                sem = sems[bsel]

                @pl.when(t > 0)
                def _():
                    pltpu.make_async_copy(
                        buf, wd_hbm.at[pl.ds(base, grp)], sem).wait()
                    for gg in range(grp):
                        row_id = jnp.full((LANES,), gg, jnp.int32)
                        old_r = (g - 2) * grp + gg
                        for c in range(nchunk):
                            idx = cols_v[old_r, pl.ds(c * LANES, LANES)]
                            plsc.store_scatter(buf, [row_id, idx], zero16)

                for gg in range(grp):
                    row_id = jnp.full((LANES,), gg, jnp.int32)
                    r = g * grp + gg
                    for c in range(nchunk):
                        idx = cols_v[r, pl.ds(c * LANES, LANES)]
                        v = vals_v[r, pl.ds(c * LANES, LANES)]
                        plsc.addupdate_scatter(buf, [row_id, idx], v)

                pltpu.async_copy(buf, wd_hbm.at[pl.ds(base + g * grp, grp)],
                                 sem)
            return carry

        lax.fori_loop(0, ngroups // 2, pair_body, 0)
        pltpu.make_async_copy(buf0, wd_hbm.at[pl.ds(base, grp)], sem0).wait()
        pltpu.make_async_copy(buf1, wd_hbm.at[pl.ds(base, grp)], sem1).wait()

    return scatter_kernel(vals, cols)


def _matmul_body(x_ref, wd_ref, bias_ref, out_ref):
    acc = lax.dot_general(
        x_ref[...], wd_ref[...],
        dimension_numbers=(((1,), (1,)), ((), ())),
        preferred_element_type=jnp.float32,
    )
    nb = out_ref.shape[1]
    j = pl.program_id(0)
    out_ref[...] = acc + bias_ref[pl.ds(j * nb, nb)][None, :]


def kernel(input, W_val, W_cols, bias):
    b, m = input.shape
    n, k = W_val.shape
    kp = ((k + LANES - 1) // LANES) * LANES
    vals = jnp.pad(W_val, ((0, 0), (0, kp - k)))
    cols = jnp.pad(W_cols.astype(jnp.int32), ((0, 0), (0, kp - k)))

    wd = _build_dense(vals, cols, n, m)

    nb = 512
    out = pl.pallas_call(
        _matmul_body,
        grid=(n // nb,),
        in_specs=[
            pl.BlockSpec((b, m), lambda i: (0, 0)),
            pl.BlockSpec((nb, m), lambda i: (i, 0)),
            pl.BlockSpec((n,), lambda i: (0,)),
        ],
        out_specs=pl.BlockSpec((b, nb), lambda i: (0, i)),
        out_shape=jax.ShapeDtypeStruct((b, n), jnp.float32),
        compiler_params=pltpu.CompilerParams(
            dimension_semantics=("arbitrary",)),
    )(input, wd, bias)
    return out
